# Initial kernel scaffold; baseline (speedup 1.0000x reference)
import jax
import jax.numpy as jnp
from jax.experimental import pallas as pl


def kernel(x, embed_weight, concepts):
    raise NotImplementedError("write your pallas kernel here")


# speedup vs baseline: 3.8210x; 3.8210x over previous
import functools

import jax
import jax.numpy as jnp
from jax import lax
from jax.experimental import pallas as pl
from jax.experimental.pallas import tpu as pltpu, tpu_sc as plsc

VOCAB = 100000
DIM = 64
NUM_CORES = 2
NUM_SUBCORES = 16
NUM_WORKERS = NUM_CORES * NUM_SUBCORES
GROUP = 128


def _make_gather(total_rows: int):
    assert total_rows % (NUM_WORKERS * GROUP) == 0
    groups_per_worker = total_rows // (NUM_WORKERS * GROUP)
    mesh = plsc.VectorSubcoreMesh(core_axis_name="c", subcore_axis_name="s")

    @functools.partial(
        pl.kernel,
        out_type=jax.ShapeDtypeStruct((total_rows // GROUP, GROUP, DIM),
                                      jnp.float32),
        mesh=mesh,
        compiler_params=pltpu.CompilerParams(use_tc_tiling_on_sc=False),
        scratch_types=[
            pltpu.VMEM((groups_per_worker, GROUP), jnp.int32),
            pltpu.VMEM((2, GROUP, DIM), jnp.float32),
            pltpu.SemaphoreType.DMA,
        ],
    )
    def gather_kernel(table_hbm, idx_hbm, out_hbm, idx_v, rows_v, gsem):
        wid = lax.axis_index("s") * NUM_CORES + lax.axis_index("c")
        gbase = wid * groups_per_worker
        pltpu.sync_copy(idx_hbm.at[wid], idx_v)

        pltpu.async_copy(table_hbm.at[idx_v.at[0]], rows_v.at[0], gsem)

        def step(j, _):
            slot = lax.rem(j, 2)
            pltpu.make_async_copy(table_hbm.at[idx_v.at[j]],
                                  rows_v.at[slot], gsem).wait()
            @pl.when(j + 1 < groups_per_worker)
            def _fire_next():
                pltpu.async_copy(table_hbm.at[idx_v.at[j + 1]],
                                 rows_v.at[1 - slot], gsem)
            pltpu.sync_copy(rows_v.at[slot], out_hbm.at[gbase + j])
            return _

        lax.fori_loop(0, groups_per_worker, step, None)

    return gather_kernel


_gather = _make_gather(4096 * 50)


def kernel(x, embed_weight, concepts):
    b, h = x.shape
    table = jnp.concatenate([embed_weight, concepts.astype(jnp.float32)],
                            axis=0)
    idx = x.reshape(NUM_WORKERS, b * h // (NUM_WORKERS * GROUP),
                    GROUP).astype(jnp.int32)
    out = _gather(table, idx)
    return out.reshape(b, h, DIM)


# speedup vs baseline: 4.2259x; 1.1060x over previous
import functools

import jax
import jax.numpy as jnp
from jax import lax
from jax.experimental import pallas as pl
from jax.experimental.pallas import tpu as pltpu, tpu_sc as plsc

VOCAB = 100000
DIM = 64
NUM_CORES = 2
NUM_SUBCORES = 16
NUM_WORKERS = NUM_CORES * NUM_SUBCORES
GROUP = 128
LANES = 16
VPG = GROUP // LANES


def _make_gather(total_rows: int):
    assert total_rows % (NUM_WORKERS * GROUP) == 0
    groups_per_worker = total_rows // (NUM_WORKERS * GROUP)
    mesh = plsc.VectorSubcoreMesh(core_axis_name="c", subcore_axis_name="s")

    @functools.partial(
        pl.kernel,
        out_type=jax.ShapeDtypeStruct((total_rows // GROUP, GROUP, DIM),
                                      jnp.float32),
        mesh=mesh,
        compiler_params=pltpu.CompilerParams(use_tc_tiling_on_sc=False,
                                             needs_layout_passes=False),
        scratch_types=[
            pltpu.VMEM((groups_per_worker, GROUP), jnp.int32),
            pltpu.VMEM((2, GROUP), jnp.int32),
            pltpu.VMEM((2, GROUP, DIM), jnp.float32),
            pltpu.VMEM((1, DIM), jnp.float32),
            pltpu.SemaphoreType.DMA,
        ],
    )
    def gather_kernel(table_hbm, conc_hbm, idx_hbm, out_hbm,
                      idx_v, xidx_v, rows_v, conc_v, gsem):
        wid = lax.axis_index("s") * NUM_CORES + lax.axis_index("c")
        gbase = wid * groups_per_worker
        pltpu.sync_copy(idx_hbm.at[wid], idx_v)
        pltpu.sync_copy(conc_hbm, conc_v)
        lane = lax.iota(jnp.int32, LANES)

        def remap(j, slot):
            any_flag = None
            for v in range(VPG):
                vec = idx_v[j, pl.ds(v * LANES, LANES)]
                m = vec == VOCAB
                g = jnp.max(jnp.where(m, 1, 0))
                any_flag = g if any_flag is None else jnp.maximum(any_flag, g)
                xidx_v[slot, pl.ds(v * LANES, LANES)] = jnp.where(m, 0, vec)
            return any_flag

        def patch(j, slot):
            for v in range(VPG):
                vec = idx_v[j, pl.ds(v * LANES, LANES)]
                mi = jnp.where(vec == VOCAB, 1, 0)

                @pl.when(jnp.max(mi) > 0)
                def _vreg_fix():
                    for r in range(LANES):
                        @pl.when(jnp.max(jnp.where(lane == r, mi, 0)) > 0)
                        def _row_fix():
                            for d in range(DIM // LANES):
                                rows_v[slot, v * LANES + r,
                                       pl.ds(d * LANES, LANES)] = (
                                    conc_v[0, pl.ds(d * LANES, LANES)])

        any0 = remap(0, 0)
        pltpu.async_copy(table_hbm.at[xidx_v.at[0]], rows_v.at[0], gsem)

        def step(j, any_j):
            slot = lax.rem(j, 2)
            nxt = jnp.minimum(j + 1, groups_per_worker - 1)
            any_next = remap(nxt, 1 - slot)
            pltpu.make_async_copy(table_hbm.at[xidx_v.at[slot]],
                                  rows_v.at[slot], gsem).wait()

            @pl.when(j + 1 < groups_per_worker)
            def _fire_next():
                pltpu.async_copy(table_hbm.at[xidx_v.at[1 - slot]],
                                 rows_v.at[1 - slot], gsem)

            @pl.when(any_j > 0)
            def _patch():
                patch(j, slot)

            pltpu.sync_copy(rows_v.at[slot], out_hbm.at[gbase + j])
            return any_next

        lax.fori_loop(0, groups_per_worker, step, any0)

    return gather_kernel


_gather = _make_gather(4096 * 50)


def kernel(x, embed_weight, concepts):
    b, h = x.shape
    idx = x.reshape(NUM_WORKERS, b * h // (NUM_WORKERS * GROUP),
                    GROUP).astype(jnp.int32)
    out = _gather(embed_weight, concepts.astype(jnp.float32), idx)
    return out.reshape(b, h, DIM)


# speedup vs baseline: 4.5303x; 1.0720x over previous
import functools

import jax
import jax.numpy as jnp
from jax import lax
from jax.experimental import pallas as pl
from jax.experimental.pallas import tpu as pltpu, tpu_sc as plsc

VOCAB = 100000
DIM = 64
NUM_CORES = 2
NUM_SUBCORES = 16
NUM_WORKERS = NUM_CORES * NUM_SUBCORES
GROUP = 128
LANES = 16
VPG = GROUP // LANES
NBUF = 5


def _make_gather(total_rows: int):
    assert total_rows % (NUM_WORKERS * GROUP) == 0
    groups = total_rows // (NUM_WORKERS * GROUP)
    assert groups % NBUF == 0 and groups >= NBUF
    rounds = groups // NBUF
    mesh = plsc.VectorSubcoreMesh(core_axis_name="c", subcore_axis_name="s")

    @functools.partial(
        pl.kernel,
        out_type=jax.ShapeDtypeStruct((total_rows // GROUP, GROUP, DIM),
                                      jnp.float32),
        mesh=mesh,
        compiler_params=pltpu.CompilerParams(use_tc_tiling_on_sc=False,
                                             needs_layout_passes=False),
        scratch_types=[
            pltpu.VMEM((groups, GROUP), jnp.int32),
            pltpu.VMEM((NBUF, GROUP), jnp.int32),
            pltpu.VMEM((NBUF, GROUP, DIM), jnp.float32),
            pltpu.VMEM((1, DIM), jnp.float32),
            pltpu.SemaphoreType.DMA((NBUF,)),
            pltpu.SemaphoreType.DMA((NBUF,)),
        ],
    )
    def gather_kernel(table_hbm, conc_hbm, idx_hbm, out_hbm,
                      idx_v, xidx_v, rows_v, conc_v, gsem, osem):
        wid = lax.axis_index("s") * NUM_CORES + lax.axis_index("c")
        gbase = wid * groups
        pltpu.sync_copy(idx_hbm.at[wid], idx_v)
        pltpu.sync_copy(conc_hbm, conc_v)
        lane = lax.iota(jnp.int32, LANES)

        def remap(j, slot):
            for v in range(VPG):
                vec = idx_v[j, pl.ds(v * LANES, LANES)]
                m = vec == VOCAB
                xidx_v[slot, pl.ds(v * LANES, LANES)] = jnp.where(m, 0, vec)

        def patch(j, slot):
            for v in range(VPG):
                vec = idx_v[j, pl.ds(v * LANES, LANES)]
                mi = jnp.where(vec == VOCAB, 1, 0)

                @pl.when(jnp.max(mi) > 0)
                def _vreg_fix():
                    def row_fix(r, _):
                        @pl.when(jnp.max(jnp.where(lane == r, mi, 0)) > 0)
                        def _():
                            for d in range(DIM // LANES):
                                rows_v[slot, v * LANES + r,
                                       pl.ds(d * LANES, LANES)] = (
                                    conc_v[0, pl.ds(d * LANES, LANES)])
                        return _
                    lax.fori_loop(0, LANES, row_fix, None)

        def fire_gather(j, slot):
            pltpu.async_copy(table_hbm.at[xidx_v.at[slot]],
                             rows_v.at[slot], gsem.at[slot])

        for b in range(NBUF - 1):
            remap(b, b)
            fire_gather(b, b)

        def round_step(r, _):
            for b in range(NBUF):
                g = r * NBUF + b
                nslot = (b - 1) % NBUF
                pltpu.make_async_copy(table_hbm.at[xidx_v.at[b]],
                                      rows_v.at[b], gsem.at[b]).wait()
                patch(g, b)
                pltpu.async_copy(rows_v.at[b], out_hbm.at[gbase + g],
                                 osem.at[b])

                @pl.when(g + NBUF - 1 < groups)
                def _fire_next():
                    @pl.when(g > 0)
                    def _drain_store():
                        pltpu.make_async_copy(
                            rows_v.at[nslot], out_hbm.at[gbase + g - 1],
                            osem.at[nslot]).wait()
                    remap(g + NBUF - 1, nslot)
                    fire_gather(g + NBUF - 1, nslot)
            return _

        lax.fori_loop(0, rounds, round_step, None)
        for b in range(NBUF):
            pltpu.make_async_copy(rows_v.at[b],
                                  out_hbm.at[gbase + groups - NBUF + b],
                                  osem.at[b]).wait()

    return gather_kernel


_gather = _make_gather(4096 * 50)


def kernel(x, embed_weight, concepts):
    b, h = x.shape
    idx = x.reshape(NUM_WORKERS, b * h // (NUM_WORKERS * GROUP),
                    GROUP).astype(jnp.int32)
    out = _gather(embed_weight, concepts.astype(jnp.float32), idx)
    return out.reshape(b, h, DIM)
